# baseline (device time: 49407 ns/iter reference)
import functools
import os

import jax
import jax.numpy as jnp
from jax import lax
from jax.experimental import pallas as pl
from jax.experimental.pallas import tpu as pltpu

NO_COMM = os.environ.get("NO_COMM") == "1"
NO_KV = NO_COMM or os.environ.get("NO_KV") == "1"
NO_OG = NO_COMM or os.environ.get("NO_OG") == "1"

N_DEV = 4
SQ = 1024
QP = SQ // N_DEV
KW = 512
PIECE = 64
NP = 256 // PIECE
HQ = 8
DH = 128
D = HQ * DH
WIN = 128
SCALE = 0.08838834764831843
KV_DTYPE = jnp.int8
S_Q = 5.0 / 127


def kernel(x, Wq, K_ext, V_ext, Wo):
    my_pos_top = lax.axis_index("i")
    xq_in = lax.dynamic_slice(
        x[0], (my_pos_top * QP, 0), (QP, D)).astype(jnp.bfloat16)

    def body(xq_ref, wq_ref, k_ref, v_ref, wo_ref, out_ref,
             kv_loc, kv_f32, need,
             kv_send, kv_recv, og_send, og_recv, loc_sem):
        def quant(xf32):
            return jnp.clip(jnp.round(xf32 * (1.0 / S_Q)),
                            -127, 127).astype(jnp.int8)
        my_pos = lax.axis_index("i")
        right = lax.rem(my_pos + 1, N_DEV)
        left = lax.rem(my_pos + N_DEV - 1, N_DEV)
        opp = lax.rem(my_pos + 2, N_DEV)

        if not NO_COMM:
            barrier_sem = pltpu.get_barrier_semaphore()
            for peer in (left, right, opp):
                pl.semaphore_signal(barrier_sem, inc=1, device_id=(peer,),
                                    device_id_type=pl.DeviceIdType.MESH)
            pl.semaphore_wait(barrier_sem, 3)

        def copy(src, dst, ssem, rsem, dev):
            return pltpu.make_async_remote_copy(
                src_ref=src, dst_ref=dst, send_sem=ssem, recv_sem=rsem,
                device_id=(dev,), device_id_type=pl.DeviceIdType.MESH)

        b = [copy(kv_loc.at[:, pl.ds(384 + PIECE * i, PIECE)],
                  need.at[:, pl.ds(256 + PIECE * i, PIECE)],
                  kv_send.at[i], kv_recv.at[i], right) for i in range(NP)]
        a = copy(kv_loc.at[:, pl.ds(128, 256)], need.at[:, pl.ds(0, 256)],
                 kv_send.at[4], kv_recv.at[4], right)
        c = [copy(kv_loc.at[:, pl.ds(640 + PIECE * i, PIECE)],
                  need.at[:, pl.ds(PIECE * i, PIECE)],
                  kv_send.at[5 + i], kv_recv.at[i], left) for i in range(NP)]
        d = copy(kv_loc.at[:, pl.ds(896, 128)], need.at[:, pl.ds(256, 128)],
                 kv_send.at[9], kv_recv.at[4], left)
        t5 = copy(kv_loc.at[:, pl.ds(0, 128)], need.at[:, pl.ds(384, 128)],
                  kv_send.at[0], kv_recv.at[5], opp)
        f = [copy(need.at[:, pl.ds(256 + PIECE * i, PIECE)],
                  need.at[:, pl.ds(PIECE * i, PIECE)],
                  kv_send.at[1 + i], kv_recv.at[i], right) for i in range(NP)]
        g = [copy(need.at[:, pl.ds(PIECE * i, PIECE)],
                  need.at[:, pl.ds(256 + PIECE * i, PIECE)],
                  kv_send.at[i], kv_recv.at[6 + i], left) for i in range(NP)]

        if not NO_KV:
            @pl.when(my_pos == 0)
            def _():
                lk = pltpu.make_async_copy(k_ref, kv_f32.at[0], loc_sem.at[0])
                lv = pltpu.make_async_copy(v_ref, kv_f32.at[1], loc_sem.at[1])
                lk.start()
                lv.start()
                lk.wait()
                lv.wait()
                kv_loc[0] = quant(kv_f32[0, 0].reshape(SQ, D))
                kv_loc[1] = quant(kv_f32[1, 0].reshape(SQ, D))
                for i in range(NP):
                    b[i].start()
                a.start()
                for i in range(NP):
                    c[i].start()
                d.start()
                need[...] = kv_loc[:, 0:KW]

            @pl.when(my_pos == 1)
            def _():
                lk = pltpu.make_async_copy(k_ref.at[:, pl.ds(0, 128)],
                                           kv_f32.at[0, :, pl.ds(0, 128)],
                                           loc_sem.at[0])
                lv = pltpu.make_async_copy(v_ref.at[:, pl.ds(0, 128)],
                                           kv_f32.at[1, :, pl.ds(0, 128)],
                                           loc_sem.at[1])
                lk.start()
                lv.start()
                lk.wait()
                lv.wait()
                kv_loc[0, 0:128] = quant(kv_f32[0, 0, 0:128].reshape(128, D))
                kv_loc[1, 0:128] = quant(kv_f32[1, 0, 0:128].reshape(128, D))
                t5.start()

        q = (jnp.dot(xq_ref[...], wq_ref[...].astype(jnp.bfloat16),
                     preferred_element_type=jnp.float32)
             * (SCALE * S_Q)).astype(jnp.bfloat16)
        wob = (wo_ref[...] * S_Q).astype(jnp.bfloat16)

        if not NO_KV:
            @pl.when(my_pos == 1)
            def _():
                for i in range(NP):
                    b[i].wait_recv()
                    f[i].start()
                a.wait_recv()

            @pl.when(my_pos == 3)
            def _():
                for i in range(NP):
                    c[i].wait_recv()
                    g[i].start()
                d.wait_recv()
                t5.wait_recv()

            @pl.when(my_pos == 2)
            def _():
                for i in range(NP):
                    f[i].wait_recv()
                for i in range(NP):
                    g[i].wait_recv()

        base = jnp.maximum(0, QP * my_pos - WIN)
        k_win = need[0].astype(jnp.bfloat16)
        v_win = need[1].astype(jnp.bfloat16)
        QH = QP // 2
        og = []
        for half in range(2):
            qi_g = (QP * my_pos + half * QH
                    + lax.broadcasted_iota(jnp.int32, (QH, KW), 0))
            ki_g = base + lax.broadcasted_iota(jnp.int32, (QH, KW), 1)
            mask = jnp.abs(qi_g - ki_g) <= WIN
            ctx_heads = []
            for h in range(HQ):
                s = lax.dot_general(
                    q[half * QH:(half + 1) * QH, h * DH:(h + 1) * DH],
                    k_win[:, h * DH:(h + 1) * DH],
                    (((1,), (1,)), ((), ())),
                    preferred_element_type=jnp.float32)
                w = jnp.exp(jnp.where(mask, s, -1e9))
                w = w / jnp.sum(w, axis=-1, keepdims=True)
                ctx_heads.append(lax.dot_general(
                    w.astype(jnp.bfloat16), v_win[:, h * DH:(h + 1) * DH],
                    (((1,), (0,)), ((), ())),
                    preferred_element_type=jnp.float32))
            ctx = jnp.concatenate(ctx_heads, axis=-1).astype(jnp.bfloat16)
            half_out = jnp.dot(ctx, wob, preferred_element_type=jnp.float32)
            row0 = my_pos * QP + half * QH
            out_ref[0, pl.ds(row0, QH), :] = half_out.astype(jnp.bfloat16)

            if not NO_OG:
                mine = out_ref.at[0, pl.ds(row0, QH), :]
                for j, dev in enumerate((left, right, opp)):
                    o = copy(mine, mine, og_send.at[3 * half + j],
                             og_recv.at[3 * half + j], dev)
                    o.start()
                    og.append(o)

        if not NO_OG:
            for o in og:
                o.wait_recv()

        if not NO_KV:
            @pl.when(my_pos == 0)
            def _():
                for i in range(NP):
                    b[i].wait_send()
                a.wait_send()
                for i in range(NP):
                    c[i].wait_send()
                d.wait_send()

            @pl.when(my_pos == 1)
            def _():
                t5.wait_send()
                for i in range(NP):
                    f[i].wait_send()

            @pl.when(my_pos == 3)
            def _():
                for i in range(NP):
                    g[i].wait_send()

        if not NO_OG:
            for o in og:
                o.wait_send()

        if not NO_COMM:
            @functools.partial(pl.run_scoped, sem2=pltpu.SemaphoreType.REGULAR)
            def _(sem2):
                for peer in (left, right, opp):
                    pl.semaphore_signal(sem2, inc=1, device_id=(peer,),
                                        device_id_type=pl.DeviceIdType.MESH)
                pl.semaphore_wait(sem2, 3)

    return pl.pallas_call(
        body,
        out_shape=jax.ShapeDtypeStruct((1, SQ, D), jnp.bfloat16),
        in_specs=[
            pl.BlockSpec(memory_space=pltpu.VMEM),
            pl.BlockSpec(memory_space=pltpu.VMEM),
            pl.BlockSpec(memory_space=pl.ANY),
            pl.BlockSpec(memory_space=pl.ANY),
            pl.BlockSpec(memory_space=pltpu.VMEM),
        ],
        out_specs=pl.BlockSpec(memory_space=pltpu.VMEM),
        scratch_shapes=[
            pltpu.VMEM((2, SQ, D), KV_DTYPE),
            pltpu.VMEM((2, 1, SQ, HQ, DH), jnp.float32),
            pltpu.VMEM((2, KW, D), KV_DTYPE),
            pltpu.SemaphoreType.DMA((10,)),
            pltpu.SemaphoreType.DMA((10,)),
            pltpu.SemaphoreType.DMA((6,)),
            pltpu.SemaphoreType.DMA((6,)),
            pltpu.SemaphoreType.DMA((2,)),
        ],
        compiler_params=(pltpu.CompilerParams() if NO_COMM
                         else pltpu.CompilerParams(collective_id=0)),
    )(xq_in, Wq, K_ext, V_ext, Wo)


# device time: 48019 ns/iter; 1.0289x vs baseline; 1.0289x over previous
import functools
import os

import jax
import jax.numpy as jnp
from jax import lax
from jax.experimental import pallas as pl
from jax.experimental.pallas import tpu as pltpu

NO_COMM = os.environ.get("NO_COMM") == "1"
NO_KV = NO_COMM or os.environ.get("NO_KV") == "1"
NO_OG = NO_COMM or os.environ.get("NO_OG") == "1"

N_DEV = 4
SQ = 1024
QP = SQ // N_DEV
KW = 512
PIECE = 64
NP = 256 // PIECE
HQ = 8
DH = 128
D = HQ * DH
WIN = 128
SCALE = 0.08838834764831843
KV_DTYPE = jnp.int8
S_Q = 5.0 / 127


def kernel(x, Wq, K_ext, V_ext, Wo):
    my_pos_top = lax.axis_index("i")
    xq_in = lax.dynamic_slice(
        x[0], (my_pos_top * QP, 0), (QP, D)).astype(jnp.bfloat16)

    def body(xq_ref, wq_ref, k_ref, v_ref, wo_ref, out_ref,
             kv_loc, kv_f32, need,
             kv_send, kv_recv, og_send, og_recv, loc_sem):
        def quant(xf32):
            return jnp.clip(jnp.round(xf32 * (1.0 / S_Q)),
                            -127, 127).astype(jnp.int8)
        my_pos = lax.axis_index("i")
        right = lax.rem(my_pos + 1, N_DEV)
        left = lax.rem(my_pos + N_DEV - 1, N_DEV)
        opp = lax.rem(my_pos + 2, N_DEV)

        if not NO_COMM:
            barrier_sem = pltpu.get_barrier_semaphore()
            for peer in (left, right, opp):
                pl.semaphore_signal(barrier_sem, inc=1, device_id=(peer,),
                                    device_id_type=pl.DeviceIdType.MESH)
            pl.semaphore_wait(barrier_sem, 3)

        def copy(src, dst, ssem, rsem, dev):
            return pltpu.make_async_remote_copy(
                src_ref=src, dst_ref=dst, send_sem=ssem, recv_sem=rsem,
                device_id=(dev,), device_id_type=pl.DeviceIdType.MESH)

        b = [copy(kv_loc.at[:, pl.ds(384 + PIECE * i, PIECE)],
                  need.at[:, pl.ds(256 + PIECE * i, PIECE)],
                  kv_send.at[i], kv_recv.at[i], right) for i in range(NP)]
        a = copy(kv_loc.at[:, pl.ds(128, 256)], need.at[:, pl.ds(0, 256)],
                 kv_send.at[4], kv_recv.at[4], right)
        c = [copy(kv_loc.at[:, pl.ds(640 + PIECE * i, PIECE)],
                  need.at[:, pl.ds(PIECE * i, PIECE)],
                  kv_send.at[5 + i], kv_recv.at[i], left) for i in range(NP)]
        d = copy(kv_loc.at[:, pl.ds(896, 128)], need.at[:, pl.ds(256, 128)],
                 kv_send.at[9], kv_recv.at[4], left)
        t5 = copy(kv_loc.at[:, pl.ds(0, 128)], need.at[:, pl.ds(384, 128)],
                  kv_send.at[0], kv_recv.at[5], opp)
        f = [copy(need.at[:, pl.ds(256 + PIECE * i, PIECE)],
                  need.at[:, pl.ds(PIECE * i, PIECE)],
                  kv_send.at[1 + i], kv_recv.at[i], right) for i in range(NP)]
        g = [copy(need.at[:, pl.ds(PIECE * i, PIECE)],
                  need.at[:, pl.ds(256 + PIECE * i, PIECE)],
                  kv_send.at[i], kv_recv.at[6 + i], left) for i in range(NP)]

        if not NO_KV:
            @pl.when(my_pos == 0)
            def _():
                lk = pltpu.make_async_copy(k_ref, kv_f32.at[0], loc_sem.at[0])
                lv = pltpu.make_async_copy(v_ref, kv_f32.at[1], loc_sem.at[1])
                lk.start()
                lv.start()
                lk.wait()
                lv.wait()
                kv_loc[0] = quant(kv_f32[0, 0].reshape(SQ, D))
                kv_loc[1] = quant(kv_f32[1, 0].reshape(SQ, D))
                for i in range(NP):
                    b[i].start()
                a.start()
                for i in range(NP):
                    c[i].start()
                d.start()
                need[...] = kv_loc[:, 0:KW]

            @pl.when(my_pos == 1)
            def _():
                lk = pltpu.make_async_copy(k_ref.at[:, pl.ds(0, 128)],
                                           kv_f32.at[0, :, pl.ds(0, 128)],
                                           loc_sem.at[0])
                lv = pltpu.make_async_copy(v_ref.at[:, pl.ds(0, 128)],
                                           kv_f32.at[1, :, pl.ds(0, 128)],
                                           loc_sem.at[1])
                lk.start()
                lv.start()
                lk.wait()
                lv.wait()
                kv_loc[0, 0:128] = quant(kv_f32[0, 0, 0:128].reshape(128, D))
                kv_loc[1, 0:128] = quant(kv_f32[1, 0, 0:128].reshape(128, D))
                t5.start()

        q = (jnp.dot(xq_ref[...], wq_ref[...].astype(jnp.bfloat16),
                     preferred_element_type=jnp.float32)
             * (SCALE * S_Q)).astype(jnp.bfloat16)
        wob = (wo_ref[...] * S_Q).astype(jnp.bfloat16)

        if not NO_KV:
            @pl.when(my_pos == 1)
            def _():
                for i in range(NP):
                    b[i].wait_recv()
                    f[i].start()
                a.wait_recv()

            @pl.when(my_pos == 3)
            def _():
                for i in range(NP):
                    c[i].wait_recv()
                    g[i].start()
                d.wait_recv()
                t5.wait_recv()

            @pl.when(my_pos == 2)
            def _():
                for i in range(NP):
                    f[i].wait_recv()
                for i in range(NP):
                    g[i].wait_recv()

        base = jnp.maximum(0, QP * my_pos - WIN)
        qi_g = QP * my_pos + lax.broadcasted_iota(jnp.int32, (QP, KW), 0)
        ki_g = base + lax.broadcasted_iota(jnp.int32, (QP, KW), 1)
        mask = jnp.abs(qi_g - ki_g) <= WIN

        k_win = need[0].astype(jnp.bfloat16)
        v_win = need[1].astype(jnp.bfloat16)
        ctx_heads = []
        for h in range(HQ):
            s = lax.dot_general(
                q[:, h * DH:(h + 1) * DH], k_win[:, h * DH:(h + 1) * DH],
                (((1,), (1,)), ((), ())),
                preferred_element_type=jnp.float32)
            w = jnp.exp(jnp.where(mask, s, -1e9))
            w = w / jnp.sum(w, axis=-1, keepdims=True)
            ctx_heads.append(lax.dot_general(
                w.astype(jnp.bfloat16), v_win[:, h * DH:(h + 1) * DH],
                (((1,), (0,)), ((), ())),
                preferred_element_type=jnp.float32))
        ctx = jnp.concatenate(ctx_heads, axis=-1).astype(jnp.bfloat16)
        my_out = jnp.dot(ctx, wob, preferred_element_type=jnp.float32)
        out_ref[0, pl.ds(my_pos * QP, QP), :] = my_out.astype(jnp.bfloat16)

        og = []
        if not NO_OG:
            mine = out_ref.at[0, pl.ds(my_pos * QP, QP), :]
            for j, dev in enumerate((left, right, opp)):
                o = copy(mine, mine, og_send.at[j], og_recv.at[j], dev)
                o.start()
                og.append(o)
            for o in og:
                o.wait_recv()

        if not NO_KV:
            @pl.when(my_pos == 0)
            def _():
                for i in range(NP):
                    b[i].wait_send()
                a.wait_send()
                for i in range(NP):
                    c[i].wait_send()
                d.wait_send()

            @pl.when(my_pos == 1)
            def _():
                t5.wait_send()
                for i in range(NP):
                    f[i].wait_send()

            @pl.when(my_pos == 3)
            def _():
                for i in range(NP):
                    g[i].wait_send()

        if not NO_OG:
            for o in og:
                o.wait_send()

        if not NO_COMM:
            @functools.partial(pl.run_scoped, sem2=pltpu.SemaphoreType.REGULAR)
            def _(sem2):
                for peer in (left, right, opp):
                    pl.semaphore_signal(sem2, inc=1, device_id=(peer,),
                                        device_id_type=pl.DeviceIdType.MESH)
                pl.semaphore_wait(sem2, 3)

    return pl.pallas_call(
        body,
        out_shape=jax.ShapeDtypeStruct((1, SQ, D), jnp.bfloat16),
        in_specs=[
            pl.BlockSpec(memory_space=pltpu.VMEM),
            pl.BlockSpec(memory_space=pltpu.VMEM),
            pl.BlockSpec(memory_space=pl.ANY),
            pl.BlockSpec(memory_space=pl.ANY),
            pl.BlockSpec(memory_space=pltpu.VMEM),
        ],
        out_specs=pl.BlockSpec(memory_space=pltpu.VMEM),
        scratch_shapes=[
            pltpu.VMEM((2, SQ, D), KV_DTYPE),
            pltpu.VMEM((2, 1, SQ, HQ, DH), jnp.float32),
            pltpu.VMEM((2, KW, D), KV_DTYPE),
            pltpu.SemaphoreType.DMA((10,)),
            pltpu.SemaphoreType.DMA((10,)),
            pltpu.SemaphoreType.DMA((6,)),
            pltpu.SemaphoreType.DMA((6,)),
            pltpu.SemaphoreType.DMA((2,)),
        ],
        compiler_params=(pltpu.CompilerParams() if NO_COMM
                         else pltpu.CompilerParams(collective_id=0)),
    )(xq_in, Wq, K_ext, V_ext, Wo)


# device time: 46769 ns/iter; 1.0564x vs baseline; 1.0267x over previous
import functools
import os

import jax
import jax.numpy as jnp
from jax import lax
from jax.experimental import pallas as pl
from jax.experimental.pallas import tpu as pltpu

NO_COMM = os.environ.get("NO_COMM") == "1"
NO_KV = NO_COMM or os.environ.get("NO_KV") == "1"
NO_OG = NO_COMM or os.environ.get("NO_OG") == "1"

N_DEV = 4
SQ = 1024
QP = SQ // N_DEV
KW = 512
PIECE = 64
NP = 256 // PIECE
HQ = 8
DH = 128
D = HQ * DH
WIN = 128
SCALE = 0.08838834764831843
KV_DTYPE = jnp.int8
S_Q = 5.0 / 127


def kernel(x, Wq, K_ext, V_ext, Wo):
    my_pos_top = lax.axis_index("i")
    xq_in = lax.dynamic_slice(
        x[0], (my_pos_top * QP, 0), (QP, D)).astype(jnp.bfloat16)

    def body(xq_ref, wq_ref, k_ref, v_ref, wo_ref, out_ref,
             kv_loc, kv_f32, need,
             kv_send, kv_recv, og_send, og_recv, loc_sem):
        def quant(xf32):
            return jnp.clip(jnp.round(xf32 * (1.0 / S_Q)),
                            -127, 127).astype(jnp.int8)
        my_pos = lax.axis_index("i")
        right = lax.rem(my_pos + 1, N_DEV)
        left = lax.rem(my_pos + N_DEV - 1, N_DEV)
        opp = lax.rem(my_pos + 2, N_DEV)

        if not NO_COMM:
            barrier_sem = pltpu.get_barrier_semaphore()
            for peer in (left, right, opp):
                pl.semaphore_signal(barrier_sem, inc=1, device_id=(peer,),
                                    device_id_type=pl.DeviceIdType.MESH)
            pl.semaphore_wait(barrier_sem, 3)

        def copy(src, dst, ssem, rsem, dev):
            return pltpu.make_async_remote_copy(
                src_ref=src, dst_ref=dst, send_sem=ssem, recv_sem=rsem,
                device_id=(dev,), device_id_type=pl.DeviceIdType.MESH)

        b = [copy(kv_loc.at[:, pl.ds(384 + PIECE * i, PIECE)],
                  need.at[:, pl.ds(256 + PIECE * i, PIECE)],
                  kv_send.at[i], kv_recv.at[i], right) for i in range(NP)]
        a = copy(kv_loc.at[:, pl.ds(128, 256)], need.at[:, pl.ds(0, 256)],
                 kv_send.at[4], kv_recv.at[4], right)
        c = [copy(kv_loc.at[:, pl.ds(640 + PIECE * i, PIECE)],
                  need.at[:, pl.ds(PIECE * i, PIECE)],
                  kv_send.at[5 + i], kv_recv.at[i], left) for i in range(NP)]
        d = copy(kv_loc.at[:, pl.ds(896, 128)], need.at[:, pl.ds(256, 128)],
                 kv_send.at[9], kv_recv.at[4], left)
        t5 = copy(kv_loc.at[:, pl.ds(0, 128)], need.at[:, pl.ds(384, 128)],
                  kv_send.at[0], kv_recv.at[5], opp)
        f = [copy(need.at[:, pl.ds(256 + PIECE * i, PIECE)],
                  need.at[:, pl.ds(PIECE * i, PIECE)],
                  kv_send.at[1 + i], kv_recv.at[i], right) for i in range(NP)]
        g = [copy(need.at[:, pl.ds(PIECE * i, PIECE)],
                  need.at[:, pl.ds(256 + PIECE * i, PIECE)],
                  kv_send.at[i], kv_recv.at[6 + i], left) for i in range(NP)]

        if not NO_KV:
            @pl.when(my_pos == 0)
            def _():
                lk = pltpu.make_async_copy(k_ref, kv_f32.at[0], loc_sem.at[0])
                lv = pltpu.make_async_copy(v_ref, kv_f32.at[1], loc_sem.at[1])
                lk.start()
                lv.start()
                lk.wait()
                lv.wait()
                kv_loc[0] = quant(kv_f32[0, 0].reshape(SQ, D))
                kv_loc[1] = quant(kv_f32[1, 0].reshape(SQ, D))
                for i in range(NP):
                    b[i].start()
                a.start()
                for i in range(NP):
                    c[i].start()
                d.start()
                need[...] = kv_loc[:, 0:KW]

            @pl.when(my_pos == 1)
            def _():
                lk = pltpu.make_async_copy(k_ref.at[:, pl.ds(0, 128)],
                                           kv_f32.at[0, :, pl.ds(0, 128)],
                                           loc_sem.at[0])
                lv = pltpu.make_async_copy(v_ref.at[:, pl.ds(0, 128)],
                                           kv_f32.at[1, :, pl.ds(0, 128)],
                                           loc_sem.at[1])
                lk.start()
                lv.start()
                lk.wait()
                lv.wait()
                kv_loc[0, 0:128] = quant(kv_f32[0, 0, 0:128].reshape(128, D))
                kv_loc[1, 0:128] = quant(kv_f32[1, 0, 0:128].reshape(128, D))
                t5.start()

        q = (jnp.dot(xq_ref[...], wq_ref[...].astype(jnp.bfloat16),
                     preferred_element_type=jnp.float32)
             * (SCALE * S_Q)).astype(jnp.bfloat16)
        wob = (wo_ref[...] * S_Q).astype(jnp.bfloat16)

        if not NO_KV:
            @pl.when(my_pos == 1)
            def _():
                for i in range(NP):
                    b[i].wait_recv()
                    f[i].start()
                a.wait_recv()

            @pl.when(my_pos == 3)
            def _():
                for i in range(NP):
                    c[i].wait_recv()
                    g[i].start()
                d.wait_recv()
                t5.wait_recv()

            @pl.when(my_pos == 2)
            def _():
                for i in range(NP):
                    f[i].wait_recv()
                for i in range(NP):
                    g[i].wait_recv()

        base = jnp.maximum(0, QP * my_pos - WIN)
        qi_g = QP * my_pos + lax.broadcasted_iota(jnp.int32, (QP, KW), 0)
        ki_g = base + lax.broadcasted_iota(jnp.int32, (QP, KW), 1)
        mask = jnp.abs(qi_g - ki_g) <= WIN

        k_win = need[0].astype(jnp.bfloat16)
        v_win = need[1].astype(jnp.bfloat16)
        ctx_heads = []
        for h in range(HQ):
            s = lax.dot_general(
                q[:, h * DH:(h + 1) * DH], k_win[:, h * DH:(h + 1) * DH],
                (((1,), (1,)), ((), ())),
                preferred_element_type=jnp.float32)
            w = jnp.exp(jnp.where(mask, s, -1e9))
            w = w / jnp.sum(w, axis=-1, keepdims=True)
            ctx_heads.append(lax.dot_general(
                w.astype(jnp.bfloat16), v_win[:, h * DH:(h + 1) * DH],
                (((1,), (0,)), ((), ())),
                preferred_element_type=jnp.float32))
        ctx = jnp.concatenate(ctx_heads, axis=-1).astype(jnp.bfloat16)
        my_out = jnp.dot(ctx, wob, preferred_element_type=jnp.float32)
        out_ref[0, pl.ds(my_pos * QP, QP), :] = my_out.astype(jnp.bfloat16)

        og = []
        if not NO_OG:
            mine = out_ref.at[0, pl.ds(my_pos * QP, QP), :]
            for j, dev in enumerate((left, right, opp)):
                o = copy(mine, mine, og_send.at[j], og_recv.at[j], dev)
                o.start()
                og.append(o)
            for o in og:
                o.wait_recv()

        if not NO_KV:
            @pl.when(my_pos == 0)
            def _():
                for i in range(NP):
                    b[i].wait_send()
                a.wait_send()
                for i in range(NP):
                    c[i].wait_send()
                d.wait_send()

            @pl.when(my_pos == 1)
            def _():
                t5.wait_send()
                for i in range(NP):
                    f[i].wait_send()

            @pl.when(my_pos == 3)
            def _():
                for i in range(NP):
                    g[i].wait_send()

        if not NO_OG:
            for o in og:
                o.wait_send()


    return pl.pallas_call(
        body,
        out_shape=jax.ShapeDtypeStruct((1, SQ, D), jnp.bfloat16),
        in_specs=[
            pl.BlockSpec(memory_space=pltpu.VMEM),
            pl.BlockSpec(memory_space=pltpu.VMEM),
            pl.BlockSpec(memory_space=pl.ANY),
            pl.BlockSpec(memory_space=pl.ANY),
            pl.BlockSpec(memory_space=pltpu.VMEM),
        ],
        out_specs=pl.BlockSpec(memory_space=pltpu.VMEM),
        scratch_shapes=[
            pltpu.VMEM((2, SQ, D), KV_DTYPE),
            pltpu.VMEM((2, 1, SQ, HQ, DH), jnp.float32),
            pltpu.VMEM((2, KW, D), KV_DTYPE),
            pltpu.SemaphoreType.DMA((10,)),
            pltpu.SemaphoreType.DMA((10,)),
            pltpu.SemaphoreType.DMA((6,)),
            pltpu.SemaphoreType.DMA((6,)),
            pltpu.SemaphoreType.DMA((2,)),
        ],
        compiler_params=(pltpu.CompilerParams() if NO_COMM
                         else pltpu.CompilerParams(collective_id=0)),
    )(xq_in, Wq, K_ext, V_ext, Wo)
